# baseline (speedup 1.0000x reference)
import functools
import math

import jax
import jax.numpy as jnp
from jax import lax
from jax.experimental import pallas as pl
from jax.experimental.pallas import tpu as pltpu
from jax.experimental.pallas import tpu_sc as plsc

NUM_BUCKETS = 32
MAX_DISTANCE = 128
HEADS = 12
SEQ = 2048
NDIAG = 2 * SEQ - 1
VH_PAD = 4224
TBL_COLS = 16
GSEG = 128
NC = 2
NS = 16
L = 16
NW = NC * NS
BLK_ROWS = 16
BLK_WORDS = BLK_ROWS * SEQ

ROWS_SC = 1024
ROWS_ALL = HEADS * SEQ
ROWS_PER_W = ROWS_SC // NW
BLKS_PER_W = ROWS_PER_W // BLK_ROWS

TC_IBLK = 1024
TC_BLOCKS = (ROWS_ALL - ROWS_SC) // TC_IBLK
TC_BLK0 = ROWS_SC // TC_IBLK
NGRP = TC_IBLK // 8
NROLL = 16
VSK_W = VH_PAD + 128


def _diag_bucket_table():
    rel = jnp.arange(NDIAG, dtype=jnp.int32) - (SEQ - 1)
    n = -rel
    half = NUM_BUCKETS // 2
    ret = (n < 0).astype(jnp.int32) * half
    n = jnp.abs(n)
    max_exact = half // 2
    is_small = n < max_exact
    safe_n = jnp.maximum(n, 1)
    val_if_large = max_exact + (
        jnp.log(safe_n.astype(jnp.float32) / max_exact)
        / math.log(MAX_DISTANCE / max_exact)
        * (half - max_exact)
    ).astype(jnp.int32)
    val_if_large = jnp.minimum(val_if_large, jnp.full_like(val_if_large, half - 1))
    ret = ret + jnp.where(is_small, n, val_if_large)
    return jnp.pad(ret, (0, VH_PAD - NDIAG))


def _sc_expand(bucket, table_flat):
    mesh = plsc.VectorSubcoreMesh(core_axis_name="c", subcore_axis_name="s")

    @functools.partial(
        pl.kernel,
        mesh=mesh,
        out_type=jax.ShapeDtypeStruct((ROWS_SC * SEQ,), jnp.float32),
        scratch_types=[
            pltpu.VMEM((VH_PAD,), jnp.int32),
            pltpu.VMEM((NUM_BUCKETS * TBL_COLS + L,), jnp.float32),
            pltpu.VMEM((VH_PAD,), jnp.float32),
            pltpu.VMEM((BLK_WORDS,), jnp.float32),
            pltpu.VMEM((BLK_WORDS,), jnp.float32),
            pltpu.SemaphoreType.DMA,
            pltpu.SemaphoreType.DMA,
        ],
    )
    def k(bucket_hbm, table_hbm, out_hbm,
          bucket_v, table_v, vh_v, buf0, buf1, sem0, sem1):
        wid = lax.axis_index("s") * NC + lax.axis_index("c")
        pltpu.sync_copy(bucket_hbm, bucket_v)
        pltpu.sync_copy(table_hbm, table_v)
        r_base = wid * ROWS_PER_W

        def build_vh(h):
            splats = [
                jnp.full(
                    (L,), table_v[pl.ds(b * TBL_COLS + h, L)][0], dtype=jnp.float32
                )
                for b in range(NUM_BUCKETS)
            ]

            i_base = lax.rem(r_base, SEQ)
            q_lo = lax.max(0, (SEQ - 47 - i_base)) // L
            q_hi = (2 * SEQ - 2 - i_base) // L + 1

            @plsc.parallel_loop(q_lo, q_hi, unroll=2)
            def chunk(q):
                bv = bucket_v[pl.ds(q * L, L)]
                acc = splats[0]
                for b in range(1, NUM_BUCKETS):
                    acc = jnp.where(bv == b, splats[b], acc)
                vh_v[pl.ds(q * L, L)] = acc

        def build_block(r0, buf):
            i0 = lax.rem(r0, SEQ)
            base = (SEQ - 1) - i0

            @plsc.parallel_loop(0, SEQ // L, unroll=2)
            def cols(q):
                o = q * L
                vals = [vh_v[pl.ds(base - rr + o, L)] for rr in range(BLK_ROWS)]
                for rr in range(BLK_ROWS):
                    buf[pl.ds(rr * SEQ + o, L)] = vals[rr]

        def fire(r0, buf, sem):
            pl.delay(100)
            dst = pl.multiple_of(r0 * SEQ, BLK_WORDS)
            pltpu.async_copy(buf.at[:], out_hbm.at[pl.ds(dst, BLK_WORDS)], sem)

        def wait(buf, sem):
            pltpu.make_async_copy(
                buf.at[:], out_hbm.at[pl.ds(0, BLK_WORDS)], sem
            ).wait()

        build_vh(r_base // SEQ)
        build_block(r_base, buf0)
        fire(r_base, buf0, sem0)

        def blk_body(b, carry2):
            r0 = r_base + b * BLK_ROWS

            def do(par, buf, sem):
                @pl.when(lax.rem(b, 2) == par)
                def _():
                    @pl.when(b >= 2)
                    def _w():
                        wait(buf, sem)

                    build_block(r0, buf)
                    fire(r0, buf, sem)

            do(1, buf1, sem1)
            do(0, buf0, sem0)
            return carry2

        if BLKS_PER_W > 1:
            lax.fori_loop(1, BLKS_PER_W, blk_body, 0)
            wait(buf1, sem1)
        wait(buf0, sem0)

    return k(bucket, table_flat)


def _tc_expand(bucket2d, table_t):
    def body(bucket_ref, tbl_ref, out_ref, rolled_ref):
        g = pl.program_id(0)
        row0 = ROWS_SC + g * TC_IBLK
        h = row0 // SEQ
        i0 = lax.rem(row0, SEQ)

        @pl.when(jnp.logical_or(g == 0, i0 == 0))
        def _build():
            bv = bucket_ref[...]
            vh = jnp.full((1, VH_PAD), tbl_ref[h, 0], dtype=jnp.float32)
            for b in range(1, NUM_BUCKETS):
                vh = jnp.where(bv == b, tbl_ref[h, b], vh)
            vhp = jnp.pad(vh, ((0, 0), (0, VSK_W - VH_PAD)))
            vsk = jnp.concatenate(
                [pltpu.roll(vhp, r, axis=1) for r in range(8)], axis=0
            )
            for m in range(NGRP // 2):
                bm = (127 - 8 * m) % 128
                rolled_ref[m] = pltpu.roll(vsk, VSK_W - bm, axis=1)

        for g in range(NGRP):
            m = g % NROLL
            bm = (127 - 8 * m) % 128
            s = (SEQ - 1) - i0 - 8 * g
            off = pl.multiple_of(s - bm, 128)
            out_ref[8 * g : 8 * g + 8, :] = rolled_ref[m, :, pl.ds(off, SEQ)]

    return pl.pallas_call(
        body,
        grid=(TC_BLOCKS,),
        in_specs=[
            pl.BlockSpec((1, VH_PAD), lambda g: (0, 0)),
            pl.BlockSpec(memory_space=pltpu.SMEM),
        ],
        out_specs=pl.BlockSpec((TC_IBLK, SEQ), lambda g: (TC_BLK0 + g, 0)),
        out_shape=jax.ShapeDtypeStruct((ROWS_ALL, SEQ), jnp.float32),
        scratch_shapes=[pltpu.VMEM((NROLL, 8, VSK_W), jnp.float32)],
        compiler_params=pltpu.CompilerParams(
            dimension_semantics=("arbitrary",),
        ),
    )(bucket2d, table_t)


def _paste(sc2d, tc_full):
    def body(sc_ref, full_ref, out_ref):
        out_ref[...] = sc_ref[...]

    return pl.pallas_call(
        body,
        grid=(ROWS_SC // 512,),
        in_specs=[
            pl.BlockSpec((512, SEQ), lambda g: (g, 0)),
            pl.BlockSpec(memory_space=pl.ANY),
        ],
        out_specs=pl.BlockSpec((512, SEQ), lambda g: (g, 0)),
        out_shape=jax.ShapeDtypeStruct((ROWS_ALL, SEQ), jnp.float32),
        input_output_aliases={1: 0},
    )(sc2d, tc_full)


def kernel(seq_length, table):
    del seq_length
    bucket = _diag_bucket_table()
    table_pad = jnp.pad(table, ((0, 0), (0, TBL_COLS - HEADS)))
    sc_flat = _sc_expand(bucket, jnp.pad(table_pad.reshape(-1), (0, L)))
    tc_full = _tc_expand(bucket.reshape(1, VH_PAD), table_pad.T)
    out = _paste(sc_flat.reshape(ROWS_SC, SEQ), tc_full)
    return out.reshape(HEADS, SEQ, SEQ)


# speedup vs baseline: 1.0126x; 1.0126x over previous
import functools
import math

import jax
import jax.numpy as jnp
from jax import lax
from jax.experimental import pallas as pl
from jax.experimental.pallas import tpu as pltpu
from jax.experimental.pallas import tpu_sc as plsc

NUM_BUCKETS = 32
MAX_DISTANCE = 128
HEADS = 12
SEQ = 2048
NDIAG = 2 * SEQ - 1
VH_PAD = 4224
TBL_COLS = 16
NC = 2
NS = 16
L = 16
NW = NC * NS
BLK_ROWS = 16
BLK_WORDS = BLK_ROWS * SEQ

ROWS_SC = 1024
ROWS_ALL = HEADS * SEQ
ROWS_PER_W = ROWS_SC // NW
BLKS_PER_W = ROWS_PER_W // BLK_ROWS

TC_IBLK = 1024
TC_BLOCKS = (ROWS_ALL - ROWS_SC) // TC_IBLK
TC_BLK0 = ROWS_SC // TC_IBLK
NGRP = TC_IBLK // 8
NROLL = 16
VSK_W = VH_PAD + 128


def _diag_bucket_table():
    rel = jnp.arange(NDIAG, dtype=jnp.int32) - (SEQ - 1)
    n = -rel
    half = NUM_BUCKETS // 2
    ret = (n < 0).astype(jnp.int32) * half
    n = jnp.abs(n)
    max_exact = half // 2
    is_small = n < max_exact
    safe_n = jnp.maximum(n, 1)
    val_if_large = max_exact + (
        jnp.log(safe_n.astype(jnp.float32) / max_exact)
        / math.log(MAX_DISTANCE / max_exact)
        * (half - max_exact)
    ).astype(jnp.int32)
    val_if_large = jnp.minimum(val_if_large, jnp.full_like(val_if_large, half - 1))
    ret = ret + jnp.where(is_small, n, val_if_large)
    return jnp.pad(ret, (0, VH_PAD - NDIAG))


def _sc_expand(bucket, table_flat):
    mesh = plsc.VectorSubcoreMesh(core_axis_name="c", subcore_axis_name="s")

    @functools.partial(
        pl.kernel,
        mesh=mesh,
        out_type=jax.ShapeDtypeStruct((ROWS_SC * SEQ,), jnp.float32),
        scratch_types=[
            pltpu.VMEM((VH_PAD,), jnp.int32),
            pltpu.VMEM((NUM_BUCKETS * TBL_COLS + L,), jnp.float32),
            pltpu.VMEM((VH_PAD,), jnp.float32),
            pltpu.VMEM((BLK_WORDS,), jnp.float32),
            pltpu.VMEM((BLK_WORDS,), jnp.float32),
            pltpu.SemaphoreType.DMA,
            pltpu.SemaphoreType.DMA,
        ],
    )
    def k(bucket_hbm, table_hbm, out_hbm,
          bucket_v, table_v, vh_v, buf0, buf1, sem0, sem1):
        wid = lax.axis_index("s") * NC + lax.axis_index("c")
        pltpu.sync_copy(bucket_hbm, bucket_v)
        pltpu.sync_copy(table_hbm, table_v)
        r_base = wid * ROWS_PER_W

        def build_vh(h):
            splats = [
                jnp.full(
                    (L,), table_v[pl.ds(b * TBL_COLS + h, L)][0], dtype=jnp.float32
                )
                for b in range(NUM_BUCKETS)
            ]

            i_base = lax.rem(r_base, SEQ)
            q_lo = lax.max(0, (SEQ - 47 - i_base)) // L
            q_hi = (2 * SEQ - 2 - i_base) // L + 1

            @plsc.parallel_loop(q_lo, q_hi, unroll=2)
            def chunk(q):
                bv = bucket_v[pl.ds(q * L, L)]
                acc = splats[0]
                for b in range(1, NUM_BUCKETS):
                    acc = jnp.where(bv == b, splats[b], acc)
                vh_v[pl.ds(q * L, L)] = acc

        def build_block(r0, buf):
            i0 = lax.rem(r0, SEQ)
            base = (SEQ - 1) - i0

            @plsc.parallel_loop(0, SEQ // L, unroll=2)
            def cols(q):
                o = q * L
                vals = [vh_v[pl.ds(base - rr + o, L)] for rr in range(BLK_ROWS)]
                for rr in range(BLK_ROWS):
                    buf[pl.ds(rr * SEQ + o, L)] = vals[rr]

        def fire(r0, buf, sem):
            pl.delay(100)
            dst = pl.multiple_of(r0 * SEQ, BLK_WORDS)
            pltpu.async_copy(buf.at[:], out_hbm.at[pl.ds(dst, BLK_WORDS)], sem)

        def wait(buf, sem):
            pltpu.make_async_copy(
                buf.at[:], out_hbm.at[pl.ds(0, BLK_WORDS)], sem
            ).wait()

        build_vh(r_base // SEQ)
        build_block(r_base, buf0)
        fire(r_base, buf0, sem0)

        def blk_body(b, carry2):
            r0 = r_base + b * BLK_ROWS

            def do(par, buf, sem):
                @pl.when(lax.rem(b, 2) == par)
                def _():
                    @pl.when(b >= 2)
                    def _w():
                        wait(buf, sem)

                    build_block(r0, buf)
                    fire(r0, buf, sem)

            do(1, buf1, sem1)
            do(0, buf0, sem0)
            return carry2

        if BLKS_PER_W > 1:
            lax.fori_loop(1, BLKS_PER_W, blk_body, 0)
            wait(buf1, sem1)
        wait(buf0, sem0)

    return k(bucket, table_flat)


def _tc_expand(bucket2d, table_t):
    def body(bucket_ref, tbl_ref, out_ref, rolled_ref):
        g = pl.program_id(0)
        row0 = ROWS_SC + g * TC_IBLK
        h = row0 // SEQ
        i0 = lax.rem(row0, SEQ)

        @pl.when(jnp.logical_or(g == 0, i0 == 0))
        def _build():
            bv = bucket_ref[...]
            vh = jnp.full((1, VH_PAD), tbl_ref[h, 0], dtype=jnp.float32)
            for b in range(1, NUM_BUCKETS):
                vh = jnp.where(bv == b, tbl_ref[h, b], vh)
            vhp = jnp.pad(vh, ((0, 0), (0, VSK_W - VH_PAD)))
            vsk = jnp.concatenate(
                [pltpu.roll(vhp, r, axis=1) for r in range(8)], axis=0
            )
            for m in range(NGRP // 2):
                bm = (127 - 8 * m) % 128
                rolled_ref[m] = pltpu.roll(vsk, VSK_W - bm, axis=1)

        for g in range(NGRP):
            m = g % NROLL
            bm = (127 - 8 * m) % 128
            s = (SEQ - 1) - i0 - 8 * g
            off = pl.multiple_of(s - bm, 128)
            out_ref[8 * g : 8 * g + 8, :] = rolled_ref[m, :, pl.ds(off, SEQ)]

    return pl.pallas_call(
        body,
        grid=(TC_BLOCKS,),
        in_specs=[
            pl.BlockSpec((1, VH_PAD), lambda g: (0, 0)),
            pl.BlockSpec(memory_space=pltpu.SMEM),
        ],
        out_specs=pl.BlockSpec((TC_IBLK, SEQ), lambda g: (TC_BLK0 + g, 0)),
        out_shape=jax.ShapeDtypeStruct((ROWS_ALL, SEQ), jnp.float32),
        scratch_shapes=[pltpu.VMEM((NROLL, 8, VSK_W), jnp.float32)],
        compiler_params=pltpu.CompilerParams(
            dimension_semantics=("arbitrary",),
        ),
    )(bucket2d, table_t)


def _paste(sc2d, tc_full):
    def body(sc_ref, full_ref, out_ref):
        out_ref[...] = sc_ref[...]

    return pl.pallas_call(
        body,
        grid=(ROWS_SC // 1024,),
        in_specs=[
            pl.BlockSpec((1024, SEQ), lambda g: (g, 0)),
            pl.BlockSpec(memory_space=pl.ANY),
        ],
        out_specs=pl.BlockSpec((1024, SEQ), lambda g: (g, 0)),
        out_shape=jax.ShapeDtypeStruct((ROWS_ALL, SEQ), jnp.float32),
        input_output_aliases={1: 0},
    )(sc2d, tc_full)


def kernel(seq_length, table):
    del seq_length
    bucket = _diag_bucket_table()
    table_pad = jnp.pad(table, ((0, 0), (0, TBL_COLS - HEADS)))
    sc_flat = _sc_expand(bucket, jnp.pad(table_pad.reshape(-1), (0, L)))
    tc_full = _tc_expand(bucket.reshape(1, VH_PAD), table_pad.T)
    out = _paste(sc_flat.reshape(ROWS_SC, SEQ), tc_full)
    return out.reshape(HEADS, SEQ, SEQ)
